# baseline (speedup 1.0000x reference)
import functools
import numpy as np
import jax
import jax.numpy as jnp
from jax import lax
from jax.experimental import pallas as pl
from jax.experimental.pallas import tpu as pltpu
from jax.experimental.pallas import tpu_sc as plsc


def _tf2x32(k1, k2, x0, x1):
    ks0, ks1 = k1, k2
    ks2 = k1 ^ k2 ^ np.uint32(0x1BD11BDA)
    rot1 = (13, 15, 26, 6)
    rot2 = (17, 29, 16, 24)

    def rnd(a, b, r):
        a = a + b
        b = (b << np.uint32(r)) | (b >> np.uint32(32 - r))
        b = a ^ b
        return a, b

    x0 = x0 + ks0
    x1 = x1 + ks1
    for r in rot1:
        x0, x1 = rnd(x0, x1, r)
    x0 = x0 + ks1; x1 = x1 + ks2 + np.uint32(1)
    for r in rot2:
        x0, x1 = rnd(x0, x1, r)
    x0 = x0 + ks2; x1 = x1 + ks0 + np.uint32(2)
    for r in rot1:
        x0, x1 = rnd(x0, x1, r)
    x0 = x0 + ks0; x1 = x1 + ks1 + np.uint32(3)
    for r in rot2:
        x0, x1 = rnd(x0, x1, r)
    x0 = x0 + ks1; x1 = x1 + ks2 + np.uint32(4)
    for r in rot1:
        x0, x1 = rnd(x0, x1, r)
    x0 = x0 + ks2; x1 = x1 + ks0 + np.uint32(5)
    return x0, x1


def _np_key_constants():
    k1 = np.uint32(0)
    k2 = np.uint32(42)
    chi = np.array([0, 0], dtype=np.uint32)
    clo = np.array([0, 1], dtype=np.uint32)
    b1, b2 = _tf2x32(k1, k2, chi, clo)
    return (b1[0], b2[0]), (b1[1], b2[1])


_K_NOISE, _K_RES = _np_key_constants()
_INF = np.float32(np.inf)


def _stageb_body(x_ref, n_ref, tt_ref, ct_ref, ft_ref, obs_ref, npart_ref,
                 upd_ref, nv_ref):
    b = pl.program_id(0)
    x = x_ref[...]
    n = n_ref[...]
    upd = jnp.dot(x, tt_ref[...], preferred_element_type=jnp.float32)
    upd = upd + jnp.dot(n, ct_ref[...], preferred_element_type=jnp.float32)
    pred = jnp.dot(upd, ft_ref[...], preferred_element_type=jnp.float32)
    d = obs_ref[...] - pred
    w = jnp.sum(d * d, axis=1, keepdims=True)
    bc = x.shape[0]
    g = b * bc + jax.lax.broadcasted_iota(jnp.int32, (bc, 1), 0)
    nv = jnp.where(g < npart_ref[0],
                   np.float32(-1.0) / (w + np.float32(1e-30)), -_INF)
    upd_ref[...] = jnp.concatenate(
        [upd, jnp.zeros((bc, 13), jnp.float32)], axis=1)
    nv_ref[...] = nv


def _make_stagec(n_particles, cp, br, cw):
    nch = cp // cw
    gr = n_particles // br

    def body(lo0_ref, hi0_ref, nv_ref, out_ref):
        lo0 = lo0_ref[0]
        hi0 = hi0_ref[0]
        hi1 = hi0 + np.uint32(1)
        k1 = _K_RES[0]
        k2 = _K_RES[1]
        lane = jax.lax.broadcasted_iota(jnp.uint32, (1, cw), 1)
        lane_i = jax.lax.broadcasted_iota(jnp.int32, (br, cw), 1)

        def chunk(c, carry):
            smin, sidx = carry
            base = lo0 + c.astype(jnp.uint32) * np.uint32(cw)
            lo = base + lane
            hi = jnp.where(lo < base, hi1, hi0)
            b1, b2 = _tf2x32(k1, k2, hi, lo)
            bits = b1 ^ b2
            f = jax.lax.bitcast_convert_type(
                (bits >> np.uint32(9)) | np.uint32(0x3F800000), jnp.float32)
            s = jnp.log(f - np.float32(1.0)) * nv_ref[c]
            m = s < smin
            smin = jnp.where(m, s, smin)
            sidx = jnp.where(m, c, sidx)
            return smin, sidx

        smin = jnp.full((br, cw), _INF, dtype=jnp.float32)
        sidx = jnp.zeros((br, cw), dtype=jnp.int32)
        smin, sidx = jax.lax.fori_loop(0, nch, chunk, (smin, sidx))
        rowmin = jnp.min(smin, axis=1, keepdims=True)
        jfull = sidx * np.int32(cw) + lane_i
        idx = jnp.min(jnp.where(smin == rowmin, jfull, jnp.int32(0x7FFFFFFF)),
                      axis=1)
        out_ref[0, 0, :] = idx

    call = pl.pallas_call(
        body,
        grid=(gr,),
        in_specs=[
            pl.BlockSpec((1, br, 1), lambda b: (b, 0, 0)),
            pl.BlockSpec((1, br, 1), lambda b: (b, 0, 0)),
            pl.BlockSpec((nch, 1, cw), lambda b: (0, 0, 0)),
        ],
        out_specs=pl.BlockSpec((1, 1, br), lambda b: (b, 0, 0)),
        out_shape=jax.ShapeDtypeStruct((gr, 1, br), jnp.int32),
    )
    return call, gr, nch


def _make_gather(gp):
    nw = 32
    bpw = gp // nw
    ch = 512
    nchunks = bpw // ch
    mesh = plsc.VectorSubcoreMesh(core_axis_name="c", subcore_axis_name="s")

    @functools.partial(
        pl.kernel, mesh=mesh,
        out_type=jax.ShapeDtypeStruct((nw, 16), jnp.float32),
        scratch_types=[
            pltpu.VMEM((ch,), jnp.int32),
            pltpu.VMEM((ch, 16), jnp.float32),
            pltpu.VMEM((16,), jnp.float32),
            pltpu.SemaphoreType.DMA,
        ],
        compiler_params=pltpu.CompilerParams(use_tc_tiling_on_sc=False),
    )
    def k(table_hbm, idx_hbm, out_hbm, idx_v, rows_v, acc_v, sem):
        wid = lax.axis_index("s") * 2 + lax.axis_index("c")
        base = wid * bpw

        def chunk_body(c, acc):
            pltpu.sync_copy(idx_hbm.at[pl.ds(base + c * ch, ch)], idx_v)
            pltpu.async_copy(table_hbm.at[idx_v], rows_v, sem).wait()

            def row_body(r, a):
                return a + rows_v[r]

            return lax.fori_loop(0, ch, row_body, acc)

        acc = lax.fori_loop(0, nchunks, chunk_body,
                            jnp.zeros((16,), jnp.float32))
        acc_v[...] = acc
        pltpu.sync_copy(acc_v, out_hbm.at[wid])

    return k


def kernel(inputs, state_vector, transition_matrix, process_noise_cov,
           forward_matrix):
    n = state_vector.shape[0]
    cw = 512
    br = 8
    cp = ((n + cw - 1) // cw) * cw
    if cp == n:
        cp += cw
    bc = min(cp, 2048)
    while cp % bc:
        bc //= 2

    key = jax.random.key(42)
    k_noise, _ = jax.random.split(key)
    noise_raw = jax.random.normal(k_noise, state_vector.shape,
                                  dtype=state_vector.dtype)
    chol = jnp.linalg.cholesky(process_noise_cov)

    pad = cp - n
    xp = jnp.pad(state_vector, ((0, pad), (0, 0)))
    npd = jnp.pad(noise_raw, ((0, pad), (0, 0)))

    upd16, nv = pl.pallas_call(
        _stageb_body,
        grid=(cp // bc,),
        in_specs=[
            pl.BlockSpec((bc, 3), lambda b: (b, 0)),
            pl.BlockSpec((bc, 3), lambda b: (b, 0)),
            pl.BlockSpec((3, 3), lambda b: (0, 0)),
            pl.BlockSpec((3, 3), lambda b: (0, 0)),
            pl.BlockSpec((3, 64), lambda b: (0, 0)),
            pl.BlockSpec((1, 64), lambda b: (0, 0)),
            pl.BlockSpec(memory_space=pltpu.SMEM),
        ],
        out_specs=[
            pl.BlockSpec((bc, 16), lambda b: (b, 0)),
            pl.BlockSpec((bc, 1), lambda b: (b, 0)),
        ],
        out_shape=[
            jax.ShapeDtypeStruct((cp, 16), jnp.float32),
            jax.ShapeDtypeStruct((cp, 1), jnp.float32),
        ],
    )(xp, npd, transition_matrix.T, chol.T, forward_matrix.T,
      inputs.reshape(1, 64), jnp.array([n], dtype=jnp.int32))

    rows = jnp.arange(n, dtype=jnp.uint32)
    nn = np.uint32(n)
    a = (rows >> np.uint32(12)) * nn
    b_ = (rows & np.uint32(0xFFF)) * nn
    lo0 = (a << np.uint32(12)) + b_
    c0 = (lo0 < b_).astype(jnp.uint32)
    hi0 = (a >> np.uint32(20)) + c0

    callc, gr, nch = _make_stagec(n, cp, br, cw)
    idx3 = callc(lo0.reshape(gr, br, 1), hi0.reshape(gr, br, 1),
                 nv.reshape(nch, 1, cw))
    idx = idx3.reshape(n)

    gp = -(-n // (32 * 512)) * (32 * 512)
    idx_p = jnp.concatenate(
        [idx, jnp.full((gp - n,), n, dtype=jnp.int32)])
    partials = _make_gather(gp)(upd16, idx_p)
    total = jnp.sum(partials, axis=0)
    return total[:3] / np.float32(n)
